# Initial kernel scaffold; baseline (speedup 1.0000x reference)
import functools
import math

import jax
import jax.numpy as jnp
from jax import lax
from jax.experimental import pallas as pl
from jax.experimental.pallas import tpu as pltpu
from jax.experimental.pallas import tpu_sc as plsc

N = 10000
E = 320000
H = 128
BLK = 2000
GRID = N // BLK

E2 = 2 * E
CHUNK = 125
NROW = E2 // CHUNK
NW = 32
CPW = NROW // NW
RPT = N // 16


def _gelu(x):
    return 0.5 * x * (1.0 + lax.erf(x * (1.0 / math.sqrt(2.0))))


def _tc1_body(x_ref, al_ref, e0, e1, e2, pw1, pb1, pw2, pb2,
              w, b, wt, bt, ws, bs, h0_out, x2_out, s_out):
    x = x_ref[...]
    oh0 = (x[:, 0:1] == lax.broadcasted_iota(jnp.int32, (BLK, 16), 1))
    oh1 = (x[:, 1:2] == lax.broadcasted_iota(jnp.int32, (BLK, 8), 1))
    oh2 = (x[:, 2:3] == lax.broadcasted_iota(jnp.int32, (BLK, 4), 1))
    he0 = jnp.dot(oh0.astype(jnp.float32), e0[...],
                  preferred_element_type=jnp.float32)
    he1 = jnp.dot(oh1.astype(jnp.float32), e1[...],
                  preferred_element_type=jnp.float32)
    he2 = jnp.dot(oh2.astype(jnp.float32), e2[...],
                  preferred_element_type=jnp.float32)
    al = al_ref[...]
    k2 = lax.broadcasted_iota(jnp.float32, (1, 16), 1) * 2.0
    div = jnp.exp(k2 * (-math.log(10000.0) / 32.0))
    ang = al * div
    h = jnp.concatenate([he0, he1, he2, jnp.sin(ang), jnp.cos(ang)], axis=1)
    h = _gelu(jnp.dot(h, pw1[...], preferred_element_type=jnp.float32)
              + pb1[...])
    h = jnp.dot(h, pw2[...], preferred_element_type=jnp.float32) + pb2[...]
    h0_out[...] = h
    x2_out[0] = jnp.dot(h, w[...], preferred_element_type=jnp.float32) + b[...]
    x2_out[1] = jnp.dot(h, wt[...], preferred_element_type=jnp.float32) + bt[...]
    s_out[...] = jnp.dot(h, ws[...], preferred_element_type=jnp.float32) + bs[...]


def _tc2_body(p_ref, s_ref, w, b, wt, bt, ws, bs, h_out, x2_out, s_out):
    h = _gelu(p_ref[0] + p_ref[1] + s_ref[...])
    h_out[...] = h
    x2_out[0] = jnp.dot(h, w[...], preferred_element_type=jnp.float32) + b[...]
    x2_out[1] = jnp.dot(h, wt[...], preferred_element_type=jnp.float32) + bt[...]
    s_out[...] = jnp.dot(h, ws[...], preferred_element_type=jnp.float32) + bs[...]


def _tc3_body(p_ref, s_ref, h0_ref, h1_ref, pw1, pb1, pw2, pb2, out_ref):
    h2 = _gelu(p_ref[0] + p_ref[1] + s_ref[...])
    t = (jnp.dot(h0_ref[...], pw1[0:H], preferred_element_type=jnp.float32)
         + jnp.dot(h1_ref[...], pw1[H:2 * H], preferred_element_type=jnp.float32)
         + jnp.dot(h2, pw1[2 * H:3 * H], preferred_element_type=jnp.float32)
         + pb1[...])
    out_ref[...] = (jnp.dot(_gelu(t), pw2[...],
                            preferred_element_type=jnp.float32) + pb2[...])


def _row_spec():
    return pl.BlockSpec((BLK, H), lambda i: (i, 0))


def _full(shape):
    return pl.BlockSpec(shape, lambda i: tuple(0 for _ in shape))


_W = _full((H, H))
_B = _full((1, H))
_P2 = pl.BlockSpec((2, BLK, H), lambda i: (0, i, 0))

_tc1 = pl.pallas_call(
    _tc1_body,
    grid=(GRID,),
    in_specs=[pl.BlockSpec((BLK, 3), lambda i: (i, 0)),
              pl.BlockSpec((BLK, 1), lambda i: (i, 0)),
              _full((16, 32)), _full((8, 32)), _full((4, 32)),
              _W, _B, _W, _B,
              _W, _B, _W, _B, _W, _B],
    out_specs=[_row_spec(), _P2, _row_spec()],
    out_shape=[jax.ShapeDtypeStruct((N, H), jnp.float32),
               jax.ShapeDtypeStruct((2, N, H), jnp.float32),
               jax.ShapeDtypeStruct((N, H), jnp.float32)],
)

_tc2 = pl.pallas_call(
    _tc2_body,
    grid=(GRID,),
    in_specs=[_P2, _row_spec(),
              _W, _B, _W, _B, _W, _B],
    out_specs=[_row_spec(), _P2, _row_spec()],
    out_shape=[jax.ShapeDtypeStruct((N, H), jnp.float32),
               jax.ShapeDtypeStruct((2, N, H), jnp.float32),
               jax.ShapeDtypeStruct((N, H), jnp.float32)],
)

_tc3 = pl.pallas_call(
    _tc3_body,
    grid=(GRID,),
    in_specs=[_P2, _row_spec(), _row_spec(), _row_spec(),
              _full((3 * H, H)), _B, _W, _B],
    out_specs=_row_spec(),
    out_shape=jax.ShapeDtypeStruct((N, H), jnp.float32),
)


def _sc_edge_body(x2_hbm, src_hbm, dst_hbm, out_hbm,
                  src_v, dst_v, buf0, buf1, acc, sem0, sem1):
    c = lax.axis_index("c")
    tid = lax.axis_index("s")
    wid = tid * 2 + c

    @pl.loop(0, CHUNK)
    def _zero_rows(i):
        for k in range(H // 16):
            buf0[i, pl.ds(k * 16, 16)] = jnp.zeros((16,), jnp.float32)

    for r in range(RPT // CHUNK):
        pltpu.sync_copy(buf0, acc.at[pl.ds((tid * (RPT // CHUNK) + r) * CHUNK,
                                           CHUNK)])
    plsc.subcore_barrier()

    pltpu.sync_copy(src_hbm.at[pl.ds(wid * CPW, CPW)], src_v)
    pltpu.sync_copy(dst_hbm.at[pl.ds(wid * CPW, CPW)], dst_v)

    bufs = (buf0, buf1)
    sems = (sem0, sem1)

    pltpu.async_copy(x2_hbm.at[src_v.at[0]], buf0, sem0)

    @pl.loop(0, CPW, step=2)
    def _chunks(j):
        for b2 in range(2):
            jj = j + b2

            @pl.when(jj + 1 < CPW)
            def _start_next():
                pltpu.async_copy(x2_hbm.at[src_v.at[jj + 1]],
                                 bufs[(b2 + 1) % 2], sems[(b2 + 1) % 2])

            pltpu.make_async_copy(x2_hbm.at[src_v.at[jj]],
                                  bufs[b2], sems[b2]).wait()
            pltpu.sync_copy(bufs[b2], acc.at[dst_v.at[jj]], add=True)

    plsc.subcore_barrier()
    pltpu.sync_copy(acc.at[pl.ds(tid * RPT, RPT)],
                    out_hbm.at[c, pl.ds(tid * RPT, RPT)])


_sc_edge = functools.partial(
    pl.kernel,
    out_type=jax.ShapeDtypeStruct((2, N, H), jnp.float32),
    mesh=plsc.VectorSubcoreMesh(core_axis_name="c", subcore_axis_name="s"),
    scratch_types=[
        pltpu.VMEM((CPW, CHUNK), jnp.int32),
        pltpu.VMEM((CPW, CHUNK), jnp.int32),
        pltpu.VMEM((CHUNK, H), jnp.float32),
        pltpu.VMEM((CHUNK, H), jnp.float32),
        pltpu.VMEM_SHARED((N, H), jnp.float32),
        pltpu.SemaphoreType.DMA,
        pltpu.SemaphoreType.DMA,
    ],
)(_sc_edge_body)


def kernel(x_n, edge_index, abs_level, rel_level, emb0, emb1, emb2,
           pi_w1, pi_b1, pi_w2, pi_b2,
           l0_w, l0_b, l0_wt, l0_bt, l0_ws, l0_bs,
           l1_w, l1_b, l1_wt, l1_bt, l1_ws, l1_bs,
           po_w1, po_b1, po_w2, po_b2):
    row = edge_index[0].astype(jnp.int32)
    col = edge_index[1].astype(jnp.int32)
    src2 = jnp.concatenate([col, row + N]).reshape(NROW, CHUNK)
    dst2 = jnp.concatenate([row, col]).reshape(NROW, CHUNK)

    b2 = lambda v: v.reshape(1, H)
    h0, x2, s0 = _tc1(x_n.astype(jnp.int32), abs_level,
                      emb0, emb1, emb2,
                      pi_w1, b2(pi_b1), pi_w2, b2(pi_b2),
                      l0_w, b2(l0_b), l0_wt, b2(l0_bt), l0_ws, b2(l0_bs))
    p0 = _sc_edge(x2.reshape(2 * N, H), src2, dst2)
    h1, x2b, s1 = _tc2(p0, s0,
                       l1_w, b2(l1_b), l1_wt, b2(l1_bt), l1_ws, b2(l1_bs))
    p1 = _sc_edge(x2b.reshape(2 * N, H), src2, dst2)
    out = _tc3(p1, s1, h0, h1, po_w1, b2(po_b1), po_w2, b2(po_b2))
    return out


# speedup vs baseline: 7.8468x; 7.8468x over previous
import functools
import math

import jax
import jax.numpy as jnp
from jax import lax
from jax.experimental import pallas as pl
from jax.experimental.pallas import tpu as pltpu
from jax.experimental.pallas import tpu_sc as plsc

N = 10000
E = 320000
H = 128
HH = 64
BLK = 2000
GRID = N // BLK

E2 = 2 * E
CHUNK = 125
NROW = E2 // CHUNK
NW = 32
CPW = NROW // NW
NP = 10240
RPT = NP // 16


def _gelu(x):
    return 0.5 * x * (1.0 + lax.erf(x * (1.0 / math.sqrt(2.0))))


def _dense(h, w, b):
    return jnp.dot(h, w[...], preferred_element_type=jnp.float32) + b[...]


def _tc1_body(x_ref, al_ref, e0, e1, e2, pw1, pb1, pw2, pb2,
              w, b, wt, bt, ws, bs, h0_out, xa_out, xb_out, s_out):
    x = x_ref[...]
    oh0 = (x[:, 0:1] == lax.broadcasted_iota(jnp.int32, (BLK, 16), 1))
    oh1 = (x[:, 1:2] == lax.broadcasted_iota(jnp.int32, (BLK, 8), 1))
    oh2 = (x[:, 2:3] == lax.broadcasted_iota(jnp.int32, (BLK, 4), 1))
    he0 = jnp.dot(oh0.astype(jnp.float32), e0[...],
                  preferred_element_type=jnp.float32)
    he1 = jnp.dot(oh1.astype(jnp.float32), e1[...],
                  preferred_element_type=jnp.float32)
    he2 = jnp.dot(oh2.astype(jnp.float32), e2[...],
                  preferred_element_type=jnp.float32)
    al = al_ref[...]
    k2 = lax.broadcasted_iota(jnp.int32, (1, 16), 1).astype(jnp.float32) * 2.0
    div = jnp.exp(k2 * (-math.log(10000.0) / 32.0))
    ang = al * div
    h = jnp.concatenate([he0, he1, he2, jnp.sin(ang), jnp.cos(ang)], axis=1)
    h = _gelu(_dense(h, pw1, pb1))
    h = _dense(h, pw2, pb2)
    h0_out[...] = h
    xw = _dense(h, w, b)
    xt = _dense(h, wt, bt)
    xa_out[0] = xw[:, 0:HH]
    xa_out[1] = xt[:, 0:HH]
    xb_out[0] = xw[:, HH:H]
    xb_out[1] = xt[:, HH:H]
    s_out[...] = _dense(h, ws, bs)


def _tc2_body(pa_ref, pb_ref, s_ref, w, b, wt, bt, ws, bs,
              h_out, xa_out, xb_out, s_out):
    m = jnp.concatenate([pa_ref[0] + pa_ref[1], pb_ref[0] + pb_ref[1]], axis=1)
    h = _gelu(m + s_ref[...])
    h_out[...] = h
    xw = _dense(h, w, b)
    xt = _dense(h, wt, bt)
    xa_out[0] = xw[:, 0:HH]
    xa_out[1] = xt[:, 0:HH]
    xb_out[0] = xw[:, HH:H]
    xb_out[1] = xt[:, HH:H]
    s_out[...] = _dense(h, ws, bs)


def _tc3_body(pa_ref, pb_ref, s_ref, h0_ref, h1_ref,
              pw1, pb1, pw2, pb2, out_ref):
    m = jnp.concatenate([pa_ref[0] + pa_ref[1], pb_ref[0] + pb_ref[1]], axis=1)
    h2 = _gelu(m + s_ref[...])
    t = (jnp.dot(h0_ref[...], pw1[0:H], preferred_element_type=jnp.float32)
         + jnp.dot(h1_ref[...], pw1[H:2 * H], preferred_element_type=jnp.float32)
         + jnp.dot(h2, pw1[2 * H:3 * H], preferred_element_type=jnp.float32)
         + pb1[...])
    out_ref[...] = (jnp.dot(_gelu(t), pw2[...],
                            preferred_element_type=jnp.float32) + pb2[...])


def _row_spec():
    return pl.BlockSpec((BLK, H), lambda i: (i, 0))


def _full(shape):
    return pl.BlockSpec(shape, lambda i: tuple(0 for _ in shape))


_W = _full((H, H))
_B = _full((1, H))
_P2 = pl.BlockSpec((2, BLK, HH), lambda i: (0, i, 0))
_X2 = [jax.ShapeDtypeStruct((2, N, HH), jnp.float32),
       jax.ShapeDtypeStruct((2, N, HH), jnp.float32)]
_X2_SPECS = [_P2, _P2]

_tc1 = pl.pallas_call(
    _tc1_body,
    grid=(GRID,),
    in_specs=[pl.BlockSpec((BLK, 3), lambda i: (i, 0)),
              pl.BlockSpec((BLK, 1), lambda i: (i, 0)),
              _full((16, 32)), _full((8, 32)), _full((4, 32)),
              _W, _B, _W, _B,
              _W, _B, _W, _B, _W, _B],
    out_specs=[_row_spec()] + _X2_SPECS + [_row_spec()],
    out_shape=[jax.ShapeDtypeStruct((N, H), jnp.float32)] + _X2
    + [jax.ShapeDtypeStruct((N, H), jnp.float32)],
)

_tc2 = pl.pallas_call(
    _tc2_body,
    grid=(GRID,),
    in_specs=[_P2, _P2, _row_spec(),
              _W, _B, _W, _B, _W, _B],
    out_specs=[_row_spec()] + _X2_SPECS + [_row_spec()],
    out_shape=[jax.ShapeDtypeStruct((N, H), jnp.float32)] + _X2
    + [jax.ShapeDtypeStruct((N, H), jnp.float32)],
)

_tc3 = pl.pallas_call(
    _tc3_body,
    grid=(GRID,),
    in_specs=[_P2, _P2, _row_spec(), _row_spec(), _row_spec(),
              _full((3 * H, H)), _B, _W, _B],
    out_specs=_row_spec(),
    out_shape=jax.ShapeDtypeStruct((N, H), jnp.float32),
)


def _sc_edge_body(x2_hbm, src_hbm, dst_hbm, out_hbm,
                  src_v, dst_v, buf0, buf1, zbuf, acc, sem0, sem1):
    c = lax.axis_index("c")
    tid = lax.axis_index("s")
    wid = tid * 2 + c

    @pl.loop(0, 128)
    def _zero_rows(i):
        for k in range(HH // 16):
            zbuf[i, pl.ds(k * 16, 16)] = jnp.zeros((16,), jnp.float32)

    for r in range(RPT // 128):
        pltpu.sync_copy(zbuf, acc.at[pl.ds((tid * (RPT // 128) + r) * 128,
                                           128)])
    plsc.subcore_barrier()

    pltpu.sync_copy(src_hbm.at[pl.ds(wid * CPW, CPW)], src_v)
    pltpu.sync_copy(dst_hbm.at[pl.ds(wid * CPW, CPW)], dst_v)

    bufs = (buf0, buf1)
    sems = (sem0, sem1)

    pltpu.async_copy(x2_hbm.at[src_v.at[0]], buf0, sem0)

    @pl.loop(0, CPW, step=2)
    def _chunks(j):
        for b2 in range(2):
            jj = j + b2

            @pl.when(jj + 1 < CPW)
            def _start_next():
                pltpu.async_copy(x2_hbm.at[src_v.at[jj + 1]],
                                 bufs[(b2 + 1) % 2], sems[(b2 + 1) % 2])

            pltpu.make_async_copy(x2_hbm.at[src_v.at[jj]],
                                  bufs[b2], sems[b2]).wait()
            pltpu.sync_copy(bufs[b2], acc.at[dst_v.at[jj]], add=True)

    plsc.subcore_barrier()
    pltpu.sync_copy(acc.at[pl.ds(tid * RPT, RPT)],
                    out_hbm.at[c, pl.ds(tid * RPT, RPT)])


@functools.cache
def _sc_edge():
    return pl.kernel(
        _sc_edge_body,
        out_type=jax.ShapeDtypeStruct((2, NP, HH), jnp.float32),
        mesh=plsc.VectorSubcoreMesh(core_axis_name="c", subcore_axis_name="s"),
        scratch_types=[
            pltpu.VMEM((CPW, CHUNK), jnp.int32),
            pltpu.VMEM((CPW, CHUNK), jnp.int32),
            pltpu.VMEM((CHUNK, HH), jnp.float32),
            pltpu.VMEM((CHUNK, HH), jnp.float32),
            pltpu.VMEM((128, HH), jnp.float32),
            pltpu.VMEM_SHARED((NP, HH), jnp.float32),
            pltpu.SemaphoreType.DMA,
            pltpu.SemaphoreType.DMA,
        ],
        compiler_params=pltpu.CompilerParams(use_tc_tiling_on_sc=False),
    )


def kernel(x_n, edge_index, abs_level, rel_level, emb0, emb1, emb2,
           pi_w1, pi_b1, pi_w2, pi_b2,
           l0_w, l0_b, l0_wt, l0_bt, l0_ws, l0_bs,
           l1_w, l1_b, l1_wt, l1_bt, l1_ws, l1_bs,
           po_w1, po_b1, po_w2, po_b2):
    row = edge_index[0].astype(jnp.int32)
    col = edge_index[1].astype(jnp.int32)
    src2 = jnp.concatenate([col, row + N]).reshape(NROW, CHUNK)
    dst2 = jnp.concatenate([row, col]).reshape(NROW, CHUNK)

    sc = _sc_edge()
    b2 = lambda v: v.reshape(1, H)
    h0, xa, xb, s0 = _tc1(x_n.astype(jnp.int32), abs_level,
                          emb0, emb1, emb2,
                          pi_w1, b2(pi_b1), pi_w2, b2(pi_b2),
                          l0_w, b2(l0_b), l0_wt, b2(l0_bt), l0_ws, b2(l0_bs))
    pa0 = sc(xa.reshape(2 * N, HH), src2, dst2)
    pb0 = sc(xb.reshape(2 * N, HH), src2, dst2)
    h1, xa1, xb1, s1 = _tc2(pa0, pb0, s0,
                            l1_w, b2(l1_b), l1_wt, b2(l1_bt),
                            l1_ws, b2(l1_bs))
    pa1 = sc(xa1.reshape(2 * N, HH), src2, dst2)
    pb1_ = sc(xb1.reshape(2 * N, HH), src2, dst2)
    out = _tc3(pa1, pb1_, s1, h0, h1, po_w1, b2(po_b1), po_w2, b2(po_b2))
    return out
